# baseline (speedup 1.0000x reference)
import functools

import jax
import jax.numpy as jnp
from jax import lax
from jax.experimental import pallas as pl
from jax.experimental.pallas import tpu as pltpu
from jax.experimental.pallas import tpu_sc as plsc

HIDDEN = 64
TOTAL = 4096 * 200
NUM_WORKERS = 32
PER_WORKER = TOTAL // NUM_WORKERS
CHUNK = 512
NCHUNK = PER_WORKER // CHUNK

_mesh = plsc.VectorSubcoreMesh(core_axis_name="c", subcore_axis_name="s")


@functools.partial(
    pl.kernel,
    mesh=_mesh,
    out_type=jax.ShapeDtypeStruct((TOTAL, HIDDEN), jnp.float32),
    scratch_types=[
        pltpu.VMEM((CHUNK,), jnp.int32),
        pltpu.VMEM((CHUNK, HIDDEN), jnp.float32),
        pltpu.SemaphoreType.DMA,
    ],
    compiler_params=pltpu.CompilerParams(use_tc_tiling_on_sc=False),
)
def _gather_kernel(idx_hbm, table_hbm, out_hbm, idx_v, rows_v, sem):
    wid = lax.axis_index("s") * 2 + lax.axis_index("c")
    base = wid * PER_WORKER

    def body(i, carry):
        off = base + i * CHUNK
        pltpu.sync_copy(idx_hbm.at[pl.ds(off, CHUNK)], idx_v)
        pltpu.async_copy(table_hbm.at[idx_v], rows_v, sem).wait()
        pltpu.sync_copy(rows_v, out_hbm.at[pl.ds(off, CHUNK)])
        return carry

    lax.fori_loop(0, NCHUNK, body, 0)


def kernel(input_ids, word_table):
    flat = input_ids.reshape(-1).astype(jnp.int32)
    out = _gather_kernel(flat, word_table)
    return out.reshape(input_ids.shape + (word_table.shape[1],))


# speedup vs baseline: 1.0446x; 1.0446x over previous
import functools

import jax
import jax.numpy as jnp
from jax import lax
from jax.experimental import pallas as pl
from jax.experimental.pallas import tpu as pltpu
from jax.experimental.pallas import tpu_sc as plsc

HIDDEN = 64
TOTAL = 4096 * 200
NUM_WORKERS = 32
PER_WORKER = TOTAL // NUM_WORKERS
CHUNK = 640
NCHUNK = PER_WORKER // CHUNK

_mesh = plsc.VectorSubcoreMesh(core_axis_name="c", subcore_axis_name="s")


@functools.partial(
    pl.kernel,
    mesh=_mesh,
    out_type=jax.ShapeDtypeStruct((TOTAL, HIDDEN), jnp.float32),
    scratch_types=[
        pltpu.VMEM((NCHUNK, CHUNK), jnp.int32),
        pltpu.VMEM((2, CHUNK, HIDDEN), jnp.float32),
        pltpu.SemaphoreType.DMA,
        pltpu.SemaphoreType.DMA,
    ],
    compiler_params=pltpu.CompilerParams(use_tc_tiling_on_sc=False),
)
def _gather_kernel(idx_hbm, table_hbm, out_hbm, idx_v, rows_v, gsem, ssem):
    wid = lax.axis_index("s") * 2 + lax.axis_index("c")
    base = wid * PER_WORKER

    pltpu.sync_copy(idx_hbm.at[wid], idx_v)

    def start_gather(i):
        pltpu.async_copy(table_hbm.at[idx_v.at[i]], rows_v.at[i % 2], gsem)

    def wait_gather():
        pltpu.make_async_copy(
            table_hbm.at[idx_v.at[0]], rows_v.at[0], gsem).wait()

    def start_store(i):
        pltpu.async_copy(
            rows_v.at[i % 2], out_hbm.at[pl.ds(base + i * CHUNK, CHUNK)], ssem)

    def wait_store():
        pltpu.make_async_copy(
            rows_v.at[0], out_hbm.at[pl.ds(base, CHUNK)], ssem).wait()

    start_gather(0)

    def body(i, carry):
        @pl.when(i >= 1)
        def _():
            wait_store()

        @pl.when(i + 1 < NCHUNK)
        def _():
            start_gather(i + 1)

        wait_gather()
        start_store(i)
        return carry

    lax.fori_loop(0, NCHUNK, body, 0)
    wait_store()


def kernel(input_ids, word_table):
    flat = input_ids.reshape(NUM_WORKERS, NCHUNK, CHUNK).astype(jnp.int32)
    out = _gather_kernel(flat, word_table)
    return out.reshape(input_ids.shape + (word_table.shape[1],))


# speedup vs baseline: 1.0721x; 1.0263x over previous
import functools

import jax
import jax.numpy as jnp
from jax import lax
from jax.experimental import pallas as pl
from jax.experimental.pallas import tpu as pltpu
from jax.experimental.pallas import tpu_sc as plsc

HIDDEN = 64
SEQ = 200
BATCH = 4096
TOTAL = BATCH * SEQ
NUM_WORKERS = 32
PER_WORKER = TOTAL // NUM_WORKERS
CHUNK = 512
NCHUNK = PER_WORKER // CHUNK

_mesh = plsc.VectorSubcoreMesh(core_axis_name="c", subcore_axis_name="s")


@functools.partial(
    pl.kernel,
    mesh=_mesh,
    out_type=jax.ShapeDtypeStruct((SEQ, BATCH, HIDDEN), jnp.float32),
    scratch_types=[
        pltpu.VMEM((NCHUNK, CHUNK), jnp.int32),
        pltpu.VMEM((2, CHUNK, HIDDEN), jnp.float32),
        pltpu.SemaphoreType.DMA,
        pltpu.SemaphoreType.DMA,
    ],
    compiler_params=pltpu.CompilerParams(use_tc_tiling_on_sc=False),
)
def _gather_kernel(idx_hbm, table_hbm, out_hbm, idx_v, rows_v, gsem, ssem):
    wid = lax.axis_index("s") * 2 + lax.axis_index("c")
    base = wid * PER_WORKER

    pltpu.sync_copy(idx_hbm.at[wid], idx_v)

    def start_gather(i):
        pltpu.async_copy(table_hbm.at[idx_v.at[i]], rows_v.at[i % 2], gsem)

    def wait_gather():
        pltpu.make_async_copy(
            table_hbm.at[idx_v.at[0]], rows_v.at[0], gsem).wait()

    def start_store(i):
        g = base + i * CHUNK
        pltpu.async_copy(
            rows_v.at[i % 2],
            out_hbm.at[g // BATCH, pl.ds(g % BATCH, CHUNK)],
            ssem)

    def wait_store():
        pltpu.make_async_copy(
            rows_v.at[0], out_hbm.at[0, pl.ds(0, CHUNK)], ssem).wait()

    start_gather(0)

    def body(i, carry):
        @pl.when(i >= 1)
        def _():
            wait_store()

        @pl.when(i + 1 < NCHUNK)
        def _():
            start_gather(i + 1)

        wait_gather()
        start_store(i)
        return carry

    lax.fori_loop(0, NCHUNK, body, 0)
    wait_store()


def kernel(input_ids, word_table):
    idx_t = input_ids.T.reshape(NUM_WORKERS, NCHUNK, CHUNK).astype(jnp.int32)
    out = _gather_kernel(idx_t, word_table)
    return jnp.transpose(out, (1, 0, 2))
